# baseline (speedup 1.0000x reference)
import jax
import jax.numpy as jnp
from jax import lax
from jax.experimental import pallas as pl
from jax.experimental.pallas import tpu as pltpu
from jax.experimental.pallas import tpu_sc as plsc

N = 10000
E = 320000
D_IN = 128
D_H = 128
D_OUT = 64

NC = 2
NS = 16
NW = NC * NS
B = 128
KB = 4
KD = 4
CH = 80
CH2 = 160
E_PAD = NW * CH * B
N_PAD = 10016
ZR = 632
ZR_LAST = N_PAD - (NS - 1) * ZR
HW = 64
EPS = 1e-5


def _shard(s, fn):
    @pl.when(s < NS - 1)
    def _():
        fn(s * ZR, ZR)

    @pl.when(s == NS - 1)
    def _():
        fn((NS - 1) * ZR, ZR_LAST)


def _mesh():
    return plsc.VectorSubcoreMesh(core_axis_name="c", subcore_axis_name="s")


def _sc_params():
    return pltpu.CompilerParams(use_tc_tiling_on_sc=False)


def _cvt_chunk(buf_k, cbuf):
    def row4(r4, carry):
        for dr in range(8):
            r = r4 * 8 + dr
            for h in range(HW // 32):
                x = buf_k[r, pl.ds(32 * h, 32)]
                a, b = plsc.unpack(x, format=plsc.PackFormat.INTERLEAVED)
                cbuf[r, pl.ds(32 * h, 16)] = a
                cbuf[r, pl.ds(32 * h + 16, 16)] = b
        return carry

    lax.fori_loop(0, B // 8, row4, 0)


def _ring(xs_ref, acc, src_v, dst_v, bufs, cbufs, sgs, sss, n_chunks):
    nb = n_chunks // KB
    for k in range(KB):
        pltpu.async_copy(xs_ref.at[src_v.at[k]], bufs.at[k], sgs[k])

    def step(i, carry):
        for k in range(KB):
            j = i * KB + k
            m = k % 2
            pltpu.make_async_copy(
                xs_ref.at[src_v.at[j]], bufs.at[k], sgs[k]).wait()

            @pl.when(j >= 2)
            def _(j=j, m=m):
                pltpu.make_async_copy(
                    cbufs.at[m], acc.at[dst_v.at[j - 2]], sss[m]).wait()

            _cvt_chunk(bufs.at[k], cbufs.at[m])
            pltpu.async_copy(cbufs.at[m], acc.at[dst_v.at[j]], sss[m],
                             add=True)

            @pl.when(i < nb - 1)
            def _(k=k, j=j):
                pltpu.async_copy(
                    xs_ref.at[src_v.at[j + KB]], bufs.at[k], sgs[k])

        return carry

    lax.fori_loop(0, nb, step, 0)
    for m in range(2):
        j = n_chunks - 2 + m
        pltpu.make_async_copy(
            cbufs.at[m], acc.at[dst_v.at[j]], sss[m]).wait()


def _deg_body(dst_hbm, ones_hbm, zeros_hbm, degp_hbm, dst_v, ones_v, acc,
              *sems):
    c = lax.axis_index("c")
    s = lax.axis_index("s")
    wid = s * NC + c
    pltpu.sync_copy(dst_hbm.at[wid], dst_v)
    pltpu.sync_copy(ones_hbm, ones_v)
    _shard(s, lambda off, nr: pltpu.sync_copy(
        zeros_hbm.at[pl.ds(off, nr)], acc.at[pl.ds(off, nr)]))
    plsc.subcore_barrier()

    def chunk(i, carry):
        for k in range(KD):
            j = i * KD + k

            @pl.when(i > 0)
            def _(k=k, j=j):
                pltpu.make_async_copy(
                    ones_v, acc.at[dst_v.at[j - KD]], sems[k]).wait()

            pltpu.async_copy(ones_v, acc.at[dst_v.at[j]], sems[k], add=True)
        return carry

    lax.fori_loop(0, CH // KD, chunk, 0)
    for k in range(KD):
        j = CH - KD + k
        pltpu.make_async_copy(ones_v, acc.at[dst_v.at[j]], sems[k]).wait()
    plsc.subcore_barrier()
    _shard(s, lambda off, nr: pltpu.sync_copy(
        acc.at[pl.ds(off, nr)], degp_hbm.at[c, pl.ds(off, nr)]))


def _make_deg_kernel():
    return pl.kernel(
        _deg_body,
        out_type=jax.ShapeDtypeStruct((NC, N_PAD, 16), jnp.float32),
        mesh=_mesh(),
        compiler_params=_sc_params(),
        scratch_types=[
            pltpu.VMEM((CH, B), jnp.int32),
            pltpu.VMEM((B, 16), jnp.float32),
            pltpu.VMEM_SHARED((N_PAD, 16), jnp.float32),
        ] + [pltpu.SemaphoreType.DMA] * KD,
    )


def _agg_params():
    return pltpu.CompilerParams(use_tc_tiling_on_sc=False,
                                needs_layout_passes=False)


def _make_agg_fused_kernel():
    def body(xs2_hbm, src_hbm, dst_hbm, zeros_hbm, out_hbm,
             src_v, dst_v, bufs, cbufs, acc, *sems):
        c = lax.axis_index("c")
        s = lax.axis_index("s")
        pltpu.sync_copy(src_hbm.at[c, s], src_v)
        pltpu.sync_copy(dst_hbm.at[s], dst_v)
        _shard(s, lambda off, nr: pltpu.sync_copy(
            zeros_hbm.at[pl.ds(off, nr)], acc.at[pl.ds(off, nr)]))
        plsc.subcore_barrier()
        _ring(xs2_hbm, acc, src_v, dst_v, bufs, cbufs,
              sems[:KB], sems[KB:], CH2)
        plsc.subcore_barrier()
        _shard(s, lambda off, nr: pltpu.sync_copy(
            acc.at[pl.ds(off, nr)], out_hbm.at[c, pl.ds(off, nr)]))

    return pl.kernel(
        body,
        out_type=jax.ShapeDtypeStruct((NC, N_PAD, HW), jnp.float32),
        mesh=_mesh(),
        compiler_params=_agg_params(),
        scratch_types=[
            pltpu.VMEM((CH2, B), jnp.int32),
            pltpu.VMEM((CH2, B), jnp.int32),
            pltpu.VMEM((KB, B, HW), jnp.bfloat16),
            pltpu.VMEM((2, B, HW), jnp.float32),
            pltpu.VMEM_SHARED((N_PAD, HW), jnp.float32),
        ] + [pltpu.SemaphoreType.DMA] * (KB + 2),
    )


def _dinv(degp_ref):
    deg = degp_ref[0, :, 0:1] + degp_ref[1, :, 0:1] + 1.0
    return lax.rsqrt(deg)


def _t1_body(x_ref, w_ref, degp_ref, xs_ref):
    dinv = _dinv(degp_ref)
    xw = jnp.dot(x_ref[...], w_ref[...], preferred_element_type=jnp.float32)
    xs = xw * dinv
    xs_ref[0] = xs[:, :HW]
    xs_ref[1] = xs[:, HW:]


def _bn_relu_mm(agg, xs, degp_ref, b_ref, g_ref, be_ref, w_ref):
    dinv = _dinv(degp_ref)
    ta = dinv * (agg[0] + xs[0])
    tb = dinv * (agg[1] + xs[1])
    t = jnp.concatenate([ta, tb], axis=1) + b_ref[...]
    tv = t[:N, :]
    mu = jnp.mean(tv, axis=0, keepdims=True)
    var = jnp.mean(tv * tv, axis=0, keepdims=True) - mu * mu
    h = g_ref[...] * (t - mu) * lax.rsqrt(var + EPS) + be_ref[...]
    h = jnp.maximum(h, 0.0)
    xw = jnp.dot(h, w_ref[...], preferred_element_type=jnp.float32)
    rid = lax.broadcasted_iota(jnp.int32, (N_PAD, 1), 0)
    return jnp.where(rid < N, xw * _dinv(degp_ref), 0.0)


def _mid2_body(agg_ref, xs_ref, degp_ref, b_ref, g_ref, be_ref, w_ref,
               out_ref):
    xs = _bn_relu_mm(agg_ref, xs_ref, degp_ref, b_ref, g_ref, be_ref, w_ref)
    out_ref[0] = xs[:, :HW]
    out_ref[1] = xs[:, HW:]


def _mid1_body(agg_ref, xs_ref, degp_ref, b_ref, g_ref, be_ref, w_ref,
               out_ref):
    out_ref[...] = _bn_relu_mm(agg_ref, xs_ref, degp_ref, b_ref, g_ref,
                               be_ref, w_ref)


def _fin_body(p_ref, xs_ref, degp_ref, b_ref, out_ref):
    dinv = _dinv(degp_ref)
    t = dinv * (p_ref[0] + xs_ref[...]) + b_ref[...]
    out_ref[...] = t[:N, :]


def _tc_call(body, out_shapes, n_in):
    shapes = out_shapes if isinstance(out_shapes, list) else [out_shapes]
    return pl.pallas_call(
        body,
        out_shape=[jax.ShapeDtypeStruct(s, d) for s, d in shapes],
        in_specs=[pl.BlockSpec(memory_space=pltpu.VMEM)
                  for _ in range(n_in)],
        out_specs=[pl.BlockSpec(memory_space=pltpu.VMEM) for _ in shapes],
        compiler_params=pltpu.CompilerParams(
            vmem_limit_bytes=100 * 1024 * 1024),
    )


def kernel(x, edge_index, W1, b1, g1, be1, W2, b2, g2, be2, W3, b3):
    pad = E_PAD - E
    src = jnp.concatenate([edge_index[0],
                           jnp.full((pad,), N, dtype=jnp.int32)])
    dst = jnp.concatenate([edge_index[1],
                           jnp.full((pad,), N, dtype=jnp.int32)])
    srcw = src.reshape(NW, CH, B)
    dstw = dst.reshape(NW, CH, B)
    srcs = jnp.stack([src, src + N_PAD]).reshape(NC, NS, CH2, B)
    dsts = dst.reshape(NS, CH2, B)
    xp = jnp.pad(x, ((0, N_PAD - N), (0, 0)))

    ones16 = jnp.ones((B, 16), jnp.float32)
    zeros16 = jnp.zeros((N_PAD, 16), jnp.float32)
    zerosh = jnp.zeros((N_PAD, HW), jnp.float32)

    degp = _make_deg_kernel()(dstw, ones16, zeros16)

    agg_f = _make_agg_fused_kernel()

    f32 = jnp.float32
    bf16 = jnp.bfloat16
    def ileave(xs):
        n, d = xs.shape
        t = xs.reshape(n, d // 32, 2, 16)
        t = jnp.swapaxes(t, 2, 3)
        return t.reshape(n, d).astype(bf16)

    (xs1,) = _tc_call(_t1_body, [((2, N_PAD, HW), f32)], 3)(xp, W1, degp)
    a1 = agg_f(ileave(xs1.reshape(2 * N_PAD, HW)), srcs, dsts, zerosh)
    (xs2,) = _tc_call(_mid2_body, [((2, N_PAD, HW), f32)], 7)(
        a1, xs1, degp, b1, g1, be1, W2)
    a2 = agg_f(ileave(xs2.reshape(2 * N_PAD, HW)), srcs, dsts, zerosh)
    (xs3,) = _tc_call(_mid1_body, [((N_PAD, D_OUT), f32)], 7)(
        a2, xs2, degp, b2, g2, be2, W3)
    xs3s = jnp.concatenate(
        [xs3, jnp.zeros((N_PAD, D_OUT), f32)], axis=0)
    p3 = agg_f(ileave(xs3s), srcs, dsts, zerosh)
    (out,) = _tc_call(_fin_body, [((N, D_OUT), f32)], 4)(p3, xs3, degp, b3)
    return out


# speedup vs baseline: 1.0089x; 1.0089x over previous
import jax
import jax.numpy as jnp
from jax import lax
from jax.experimental import pallas as pl
from jax.experimental.pallas import tpu as pltpu
from jax.experimental.pallas import tpu_sc as plsc

N = 10000
E = 320000
D_IN = 128
D_H = 128
D_OUT = 64

NC = 2
NS = 16
NW = NC * NS
B = 128
KB = 8
KD = 4
CH = 80
CH2 = 160
E_PAD = NW * CH * B
N_PAD = 10016
ZR = 632
ZR_LAST = N_PAD - (NS - 1) * ZR
HW = 64
EPS = 1e-5


def _shard(s, fn):
    @pl.when(s < NS - 1)
    def _():
        fn(s * ZR, ZR)

    @pl.when(s == NS - 1)
    def _():
        fn((NS - 1) * ZR, ZR_LAST)


def _mesh():
    return plsc.VectorSubcoreMesh(core_axis_name="c", subcore_axis_name="s")


def _sc_params():
    return pltpu.CompilerParams(use_tc_tiling_on_sc=False)


def _cvt_chunk(buf_k, cbuf):
    def row4(r4, carry):
        for dr in range(8):
            r = r4 * 8 + dr
            for h in range(HW // 32):
                x = buf_k[r, pl.ds(32 * h, 32)]
                a, b = plsc.unpack(x, format=plsc.PackFormat.INTERLEAVED)
                cbuf[r, pl.ds(32 * h, 16)] = a
                cbuf[r, pl.ds(32 * h + 16, 16)] = b
        return carry

    lax.fori_loop(0, B // 8, row4, 0)


def _ring(xs_ref, acc, src_v, dst_v, bufs, cbufs, sgs, sss, n_chunks):
    nb = n_chunks // KB
    for k in range(KB):
        pltpu.async_copy(xs_ref.at[src_v.at[k]], bufs.at[k], sgs[k])

    def step(i, carry):
        for k in range(KB):
            j = i * KB + k
            m = k % 2
            pltpu.make_async_copy(
                xs_ref.at[src_v.at[j]], bufs.at[k], sgs[k]).wait()

            @pl.when(j >= 2)
            def _(j=j, m=m):
                pltpu.make_async_copy(
                    cbufs.at[m], acc.at[dst_v.at[j - 2]], sss[m]).wait()

            _cvt_chunk(bufs.at[k], cbufs.at[m])
            pltpu.async_copy(cbufs.at[m], acc.at[dst_v.at[j]], sss[m],
                             add=True)

            @pl.when(i < nb - 1)
            def _(k=k, j=j):
                pltpu.async_copy(
                    xs_ref.at[src_v.at[j + KB]], bufs.at[k], sgs[k])

        return carry

    lax.fori_loop(0, nb, step, 0)
    for m in range(2):
        j = n_chunks - 2 + m
        pltpu.make_async_copy(
            cbufs.at[m], acc.at[dst_v.at[j]], sss[m]).wait()


def _deg_body(dst_hbm, ones_hbm, zeros_hbm, degp_hbm, dst_v, ones_v, acc,
              *sems):
    c = lax.axis_index("c")
    s = lax.axis_index("s")
    wid = s * NC + c
    pltpu.sync_copy(dst_hbm.at[wid], dst_v)
    pltpu.sync_copy(ones_hbm, ones_v)
    _shard(s, lambda off, nr: pltpu.sync_copy(
        zeros_hbm.at[pl.ds(off, nr)], acc.at[pl.ds(off, nr)]))
    plsc.subcore_barrier()

    def chunk(i, carry):
        for k in range(KD):
            j = i * KD + k

            @pl.when(i > 0)
            def _(k=k, j=j):
                pltpu.make_async_copy(
                    ones_v, acc.at[dst_v.at[j - KD]], sems[k]).wait()

            pltpu.async_copy(ones_v, acc.at[dst_v.at[j]], sems[k], add=True)
        return carry

    lax.fori_loop(0, CH // KD, chunk, 0)
    for k in range(KD):
        j = CH - KD + k
        pltpu.make_async_copy(ones_v, acc.at[dst_v.at[j]], sems[k]).wait()
    plsc.subcore_barrier()
    _shard(s, lambda off, nr: pltpu.sync_copy(
        acc.at[pl.ds(off, nr)], degp_hbm.at[c, pl.ds(off, nr)]))


def _make_deg_kernel():
    return pl.kernel(
        _deg_body,
        out_type=jax.ShapeDtypeStruct((NC, N_PAD, 16), jnp.float32),
        mesh=_mesh(),
        compiler_params=_sc_params(),
        scratch_types=[
            pltpu.VMEM((CH, B), jnp.int32),
            pltpu.VMEM((B, 16), jnp.float32),
            pltpu.VMEM_SHARED((N_PAD, 16), jnp.float32),
        ] + [pltpu.SemaphoreType.DMA] * KD,
    )


def _agg_params():
    return pltpu.CompilerParams(use_tc_tiling_on_sc=False,
                                needs_layout_passes=False)


def _make_agg_fused_kernel():
    def body(xs2_hbm, src_hbm, dst_hbm, zeros_hbm, out_hbm,
             src_v, dst_v, bufs, cbufs, acc, *sems):
        c = lax.axis_index("c")
        s = lax.axis_index("s")
        pltpu.sync_copy(src_hbm.at[c, s], src_v)
        pltpu.sync_copy(dst_hbm.at[s], dst_v)
        _shard(s, lambda off, nr: pltpu.sync_copy(
            zeros_hbm.at[pl.ds(off, nr)], acc.at[pl.ds(off, nr)]))
        plsc.subcore_barrier()
        _ring(xs2_hbm, acc, src_v, dst_v, bufs, cbufs,
              sems[:KB], sems[KB:], CH2)
        plsc.subcore_barrier()
        _shard(s, lambda off, nr: pltpu.sync_copy(
            acc.at[pl.ds(off, nr)], out_hbm.at[c, pl.ds(off, nr)]))

    return pl.kernel(
        body,
        out_type=jax.ShapeDtypeStruct((NC, N_PAD, HW), jnp.float32),
        mesh=_mesh(),
        compiler_params=_agg_params(),
        scratch_types=[
            pltpu.VMEM((CH2, B), jnp.int32),
            pltpu.VMEM((CH2, B), jnp.int32),
            pltpu.VMEM((KB, B, HW), jnp.bfloat16),
            pltpu.VMEM((2, B, HW), jnp.float32),
            pltpu.VMEM_SHARED((N_PAD, HW), jnp.float32),
        ] + [pltpu.SemaphoreType.DMA] * (KB + 2),
    )


def _dinv(degp_ref):
    deg = degp_ref[0, :, 0:1] + degp_ref[1, :, 0:1] + 1.0
    return lax.rsqrt(deg)


def _t1_body(x_ref, w_ref, degp_ref, xs_ref):
    dinv = _dinv(degp_ref)
    xw = jnp.dot(x_ref[...], w_ref[...], preferred_element_type=jnp.float32)
    xs = xw * dinv
    xs_ref[0] = xs[:, :HW]
    xs_ref[1] = xs[:, HW:]


def _bn_relu_mm(agg, xs, degp_ref, b_ref, g_ref, be_ref, w_ref):
    dinv = _dinv(degp_ref)
    ta = dinv * (agg[0] + xs[0])
    tb = dinv * (agg[1] + xs[1])
    t = jnp.concatenate([ta, tb], axis=1) + b_ref[...]
    tv = t[:N, :]
    mu = jnp.mean(tv, axis=0, keepdims=True)
    var = jnp.mean(tv * tv, axis=0, keepdims=True) - mu * mu
    h = g_ref[...] * (t - mu) * lax.rsqrt(var + EPS) + be_ref[...]
    h = jnp.maximum(h, 0.0)
    xw = jnp.dot(h, w_ref[...], preferred_element_type=jnp.float32)
    rid = lax.broadcasted_iota(jnp.int32, (N_PAD, 1), 0)
    return jnp.where(rid < N, xw * _dinv(degp_ref), 0.0)


def _mid2_body(agg_ref, xs_ref, degp_ref, b_ref, g_ref, be_ref, w_ref,
               out_ref):
    xs = _bn_relu_mm(agg_ref, xs_ref, degp_ref, b_ref, g_ref, be_ref, w_ref)
    out_ref[0] = xs[:, :HW]
    out_ref[1] = xs[:, HW:]


def _mid1_body(agg_ref, xs_ref, degp_ref, b_ref, g_ref, be_ref, w_ref,
               out_ref):
    out_ref[...] = _bn_relu_mm(agg_ref, xs_ref, degp_ref, b_ref, g_ref,
                               be_ref, w_ref)


def _fin_body(p_ref, xs_ref, degp_ref, b_ref, out_ref):
    dinv = _dinv(degp_ref)
    t = dinv * (p_ref[0] + xs_ref[...]) + b_ref[...]
    out_ref[...] = t[:N, :]


def _tc_call(body, out_shapes, n_in):
    shapes = out_shapes if isinstance(out_shapes, list) else [out_shapes]
    return pl.pallas_call(
        body,
        out_shape=[jax.ShapeDtypeStruct(s, d) for s, d in shapes],
        in_specs=[pl.BlockSpec(memory_space=pltpu.VMEM)
                  for _ in range(n_in)],
        out_specs=[pl.BlockSpec(memory_space=pltpu.VMEM) for _ in shapes],
        compiler_params=pltpu.CompilerParams(
            vmem_limit_bytes=100 * 1024 * 1024),
    )


def kernel(x, edge_index, W1, b1, g1, be1, W2, b2, g2, be2, W3, b3):
    pad = E_PAD - E
    src = jnp.concatenate([edge_index[0],
                           jnp.full((pad,), N, dtype=jnp.int32)])
    dst = jnp.concatenate([edge_index[1],
                           jnp.full((pad,), N, dtype=jnp.int32)])
    srcw = src.reshape(NW, CH, B)
    dstw = dst.reshape(NW, CH, B)
    srcs = jnp.stack([src, src + N_PAD]).reshape(NC, NS, CH2, B)
    dsts = dst.reshape(NS, CH2, B)
    xp = jnp.pad(x, ((0, N_PAD - N), (0, 0)))

    ones16 = jnp.ones((B, 16), jnp.float32)
    zeros16 = jnp.zeros((N_PAD, 16), jnp.float32)
    zerosh = jnp.zeros((N_PAD, HW), jnp.float32)

    degp = _make_deg_kernel()(dstw, ones16, zeros16)

    agg_f = _make_agg_fused_kernel()

    f32 = jnp.float32
    bf16 = jnp.bfloat16
    def ileave(xs):
        n, d = xs.shape
        t = xs.reshape(n, d // 32, 2, 16)
        t = jnp.swapaxes(t, 2, 3)
        return t.reshape(n, d).astype(bf16)

    (xs1,) = _tc_call(_t1_body, [((2, N_PAD, HW), f32)], 3)(xp, W1, degp)
    a1 = agg_f(ileave(xs1.reshape(2 * N_PAD, HW)), srcs, dsts, zerosh)
    (xs2,) = _tc_call(_mid2_body, [((2, N_PAD, HW), f32)], 7)(
        a1, xs1, degp, b1, g1, be1, W2)
    a2 = agg_f(ileave(xs2.reshape(2 * N_PAD, HW)), srcs, dsts, zerosh)
    (xs3,) = _tc_call(_mid1_body, [((N_PAD, D_OUT), f32)], 7)(
        a2, xs2, degp, b2, g2, be2, W3)
    xs3s = jnp.concatenate(
        [xs3, jnp.zeros((N_PAD, D_OUT), f32)], axis=0)
    p3 = agg_f(ileave(xs3s), srcs, dsts, zerosh)
    (out,) = _tc_call(_fin_body, [((N, D_OUT), f32)], 4)(p3, xs3, degp, b3)
    return out


# speedup vs baseline: 1.1531x; 1.1430x over previous
import jax
import jax.numpy as jnp
from jax import lax
from jax.experimental import pallas as pl
from jax.experimental.pallas import tpu as pltpu
from jax.experimental.pallas import tpu_sc as plsc

N = 10000
E = 320000
D_IN = 128
D_H = 128
D_OUT = 64

NC = 2
NS = 16
NW = NC * NS
B = 128
KB = 8
KD = 4
CH = 80
CH2 = 160
E_PAD = NW * CH * B
N_PAD = 10016
ZR = 632
ZR_LAST = N_PAD - (NS - 1) * ZR
HW = 64
EPS = 1e-5


def _shard(s, fn):
    @pl.when(s < NS - 1)
    def _():
        fn(s * ZR, ZR)

    @pl.when(s == NS - 1)
    def _():
        fn((NS - 1) * ZR, ZR_LAST)


def _mesh():
    return plsc.VectorSubcoreMesh(core_axis_name="c", subcore_axis_name="s")


def _sc_params():
    return pltpu.CompilerParams(use_tc_tiling_on_sc=False)


def _cvt_chunk(buf_k, cbuf):
    def row4(r4, carry):
        for dr in range(8):
            r = r4 * 8 + dr
            for h in range(HW // 32):
                x = buf_k[r, pl.ds(32 * h, 32)]
                a, b = plsc.unpack(x, format=plsc.PackFormat.INTERLEAVED)
                cbuf[r, pl.ds(32 * h, 16)] = a
                cbuf[r, pl.ds(32 * h + 16, 16)] = b
        return carry

    lax.fori_loop(0, B // 8, row4, 0)


def _ring(xs_ref, acc, src_v, dst_v, bufs, cbufs, sgs, sss, n_chunks):
    nb = n_chunks // KB
    for k in range(KB):
        pltpu.async_copy(xs_ref.at[src_v.at[k]], bufs.at[k], sgs[k])

    def step(i, carry):
        for k in range(KB):
            j = i * KB + k
            m = k % 2
            pltpu.make_async_copy(
                xs_ref.at[src_v.at[j]], bufs.at[k], sgs[k]).wait()

            @pl.when(j >= 2)
            def _(j=j, m=m):
                pltpu.make_async_copy(
                    cbufs.at[m], acc.at[dst_v.at[j - 2]], sss[m]).wait()

            _cvt_chunk(bufs.at[k], cbufs.at[m])
            pltpu.async_copy(cbufs.at[m], acc.at[dst_v.at[j]], sss[m],
                             add=True)

            @pl.when(i < nb - 1)
            def _(k=k, j=j):
                pltpu.async_copy(
                    xs_ref.at[src_v.at[j + KB]], bufs.at[k], sgs[k])

        return carry

    lax.fori_loop(0, nb, step, 0)
    for m in range(2):
        j = n_chunks - 2 + m
        pltpu.make_async_copy(
            cbufs.at[m], acc.at[dst_v.at[j]], sss[m]).wait()


def _deg_body(dst_hbm, ones_hbm, zeros_hbm, degp_hbm, dst_v, ones_v, acc,
              *sems):
    c = lax.axis_index("c")
    s = lax.axis_index("s")
    wid = s * NC + c
    pltpu.sync_copy(dst_hbm.at[wid], dst_v)
    pltpu.sync_copy(ones_hbm, ones_v)
    _shard(s, lambda off, nr: pltpu.sync_copy(
        zeros_hbm.at[pl.ds(off, nr)], acc.at[pl.ds(off, nr)]))
    plsc.subcore_barrier()

    def chunk(i, carry):
        for k in range(KD):
            j = i * KD + k

            @pl.when(i > 0)
            def _(k=k, j=j):
                pltpu.make_async_copy(
                    ones_v, acc.at[dst_v.at[j - KD]], sems[k]).wait()

            pltpu.async_copy(ones_v, acc.at[dst_v.at[j]], sems[k], add=True)
        return carry

    lax.fori_loop(0, CH // KD, chunk, 0)
    for k in range(KD):
        j = CH - KD + k
        pltpu.make_async_copy(ones_v, acc.at[dst_v.at[j]], sems[k]).wait()
    plsc.subcore_barrier()
    _shard(s, lambda off, nr: pltpu.sync_copy(
        acc.at[pl.ds(off, nr)], degp_hbm.at[c, pl.ds(off, nr)]))


def _make_deg_kernel():
    return pl.kernel(
        _deg_body,
        out_type=jax.ShapeDtypeStruct((NC, N_PAD, 16), jnp.float32),
        mesh=_mesh(),
        compiler_params=_sc_params(),
        scratch_types=[
            pltpu.VMEM((CH, B), jnp.int32),
            pltpu.VMEM((B, 16), jnp.float32),
            pltpu.VMEM_SHARED((N_PAD, 16), jnp.float32),
        ] + [pltpu.SemaphoreType.DMA] * KD,
    )


def _agg_params():
    return pltpu.CompilerParams(use_tc_tiling_on_sc=False,
                                needs_layout_passes=False)


def _make_agg_fused_kernel():
    def body(xs2_hbm, src_hbm, dst_hbm, zeros_hbm, out_hbm,
             src_v, dst_v, bufs, cbufs, acc, *sems):
        c = lax.axis_index("c")
        s = lax.axis_index("s")
        pltpu.sync_copy(src_hbm.at[c, s], src_v)
        pltpu.sync_copy(dst_hbm.at[s], dst_v)
        _shard(s, lambda off, nr: pltpu.sync_copy(
            zeros_hbm.at[pl.ds(off, nr)], acc.at[pl.ds(off, nr)]))
        plsc.subcore_barrier()
        _ring(xs2_hbm, acc, src_v, dst_v, bufs, cbufs,
              sems[:KB], sems[KB:], CH2)
        plsc.subcore_barrier()
        _shard(s, lambda off, nr: pltpu.sync_copy(
            acc.at[pl.ds(off, nr)], out_hbm.at[c, pl.ds(off, nr)]))

    return pl.kernel(
        body,
        out_type=jax.ShapeDtypeStruct((NC, N_PAD, HW), jnp.float32),
        mesh=_mesh(),
        compiler_params=_agg_params(),
        scratch_types=[
            pltpu.VMEM((CH2, B), jnp.int32),
            pltpu.VMEM((CH2, B), jnp.int32),
            pltpu.VMEM((KB, B, HW), jnp.bfloat16),
            pltpu.VMEM((2, B, HW), jnp.float32),
            pltpu.VMEM_SHARED((N_PAD, HW), jnp.float32),
        ] + [pltpu.SemaphoreType.DMA] * (KB + 2),
    )


def _make_agg_part_kernel():
    def body(xs_hbm, src_hbm, dst_hbm, zeros_hbm, part_hbm,
             src_v, dst_v, bufs, cbufs, acc, *sems):
        c = lax.axis_index("c")
        s = lax.axis_index("s")
        wid = s * NC + c
        pltpu.sync_copy(src_hbm.at[wid], src_v)
        pltpu.sync_copy(dst_hbm.at[wid], dst_v)
        _shard(s, lambda off, nr: pltpu.sync_copy(
            zeros_hbm.at[pl.ds(off, nr)], acc.at[pl.ds(off, nr)]))
        plsc.subcore_barrier()
        _ring(xs_hbm, acc, src_v, dst_v, bufs, cbufs,
              sems[:KB], sems[KB:], CH)
        plsc.subcore_barrier()
        _shard(s, lambda off, nr: pltpu.sync_copy(
            acc.at[pl.ds(off, nr)], part_hbm.at[c, pl.ds(off, nr)]))

    return pl.kernel(
        body,
        out_type=jax.ShapeDtypeStruct((NC, N_PAD, HW), jnp.float32),
        mesh=_mesh(),
        compiler_params=_agg_params(),
        scratch_types=[
            pltpu.VMEM((CH, B), jnp.int32),
            pltpu.VMEM((CH, B), jnp.int32),
            pltpu.VMEM((KB, B, HW), jnp.bfloat16),
            pltpu.VMEM((2, B, HW), jnp.float32),
            pltpu.VMEM_SHARED((N_PAD, HW), jnp.float32),
        ] + [pltpu.SemaphoreType.DMA] * (KB + 2),
    )


def _dinv(degp_ref):
    deg = degp_ref[0, :, 0:1] + degp_ref[1, :, 0:1] + 1.0
    return lax.rsqrt(deg)


def _t1_body(x_ref, w_ref, degp_ref, xs_ref):
    dinv = _dinv(degp_ref)
    xw = jnp.dot(x_ref[...], w_ref[...], preferred_element_type=jnp.float32)
    xs = xw * dinv
    xs_ref[0] = xs[:, :HW]
    xs_ref[1] = xs[:, HW:]


def _bn_relu_mm(agg, xs, degp_ref, b_ref, g_ref, be_ref, w_ref):
    dinv = _dinv(degp_ref)
    ta = dinv * (agg[0] + xs[0])
    tb = dinv * (agg[1] + xs[1])
    t = jnp.concatenate([ta, tb], axis=1) + b_ref[...]
    tv = t[:N, :]
    mu = jnp.mean(tv, axis=0, keepdims=True)
    var = jnp.mean(tv * tv, axis=0, keepdims=True) - mu * mu
    h = g_ref[...] * (t - mu) * lax.rsqrt(var + EPS) + be_ref[...]
    h = jnp.maximum(h, 0.0)
    xw = jnp.dot(h, w_ref[...], preferred_element_type=jnp.float32)
    rid = lax.broadcasted_iota(jnp.int32, (N_PAD, 1), 0)
    return jnp.where(rid < N, xw * _dinv(degp_ref), 0.0)


def _mid2_body(agg_ref, xs_ref, degp_ref, b_ref, g_ref, be_ref, w_ref,
               out_ref):
    xs = _bn_relu_mm(agg_ref, xs_ref, degp_ref, b_ref, g_ref, be_ref, w_ref)
    out_ref[0] = xs[:, :HW]
    out_ref[1] = xs[:, HW:]


def _mid1_body(agg_ref, xs_ref, degp_ref, b_ref, g_ref, be_ref, w_ref,
               out_ref):
    out_ref[...] = _bn_relu_mm(agg_ref, xs_ref, degp_ref, b_ref, g_ref,
                               be_ref, w_ref)


def _fin_body(p_ref, xs_ref, degp_ref, b_ref, out_ref):
    dinv = _dinv(degp_ref)
    t = dinv * (p_ref[0] + p_ref[1] + xs_ref[...]) + b_ref[...]
    out_ref[...] = t[:N, :]


def _tc_call(body, out_shapes, n_in):
    shapes = out_shapes if isinstance(out_shapes, list) else [out_shapes]
    return pl.pallas_call(
        body,
        out_shape=[jax.ShapeDtypeStruct(s, d) for s, d in shapes],
        in_specs=[pl.BlockSpec(memory_space=pltpu.VMEM)
                  for _ in range(n_in)],
        out_specs=[pl.BlockSpec(memory_space=pltpu.VMEM) for _ in shapes],
        compiler_params=pltpu.CompilerParams(
            vmem_limit_bytes=100 * 1024 * 1024),
    )


def kernel(x, edge_index, W1, b1, g1, be1, W2, b2, g2, be2, W3, b3):
    pad = E_PAD - E
    src = jnp.concatenate([edge_index[0],
                           jnp.full((pad,), N, dtype=jnp.int32)])
    dst = jnp.concatenate([edge_index[1],
                           jnp.full((pad,), N, dtype=jnp.int32)])
    srcw = src.reshape(NW, CH, B)
    dstw = dst.reshape(NW, CH, B)
    srcs = jnp.stack([src, src + N_PAD]).reshape(NC, NS, CH2, B)
    dsts = dst.reshape(NS, CH2, B)
    xp = jnp.pad(x, ((0, N_PAD - N), (0, 0)))

    ones16 = jnp.ones((B, 16), jnp.float32)
    zeros16 = jnp.zeros((N_PAD, 16), jnp.float32)
    zerosh = jnp.zeros((N_PAD, HW), jnp.float32)

    degp = _make_deg_kernel()(dstw, ones16, zeros16)

    agg_f = _make_agg_fused_kernel()

    f32 = jnp.float32
    bf16 = jnp.bfloat16
    def ileave(xs):
        n, d = xs.shape
        t = xs.reshape(n, d // 32, 2, 16)
        t = jnp.swapaxes(t, 2, 3)
        return t.reshape(n, d).astype(bf16)

    (xs1,) = _tc_call(_t1_body, [((2, N_PAD, HW), f32)], 3)(xp, W1, degp)
    a1 = agg_f(ileave(xs1.reshape(2 * N_PAD, HW)), srcs, dsts, zerosh)
    (xs2,) = _tc_call(_mid2_body, [((2, N_PAD, HW), f32)], 7)(
        a1, xs1, degp, b1, g1, be1, W2)
    a2 = agg_f(ileave(xs2.reshape(2 * N_PAD, HW)), srcs, dsts, zerosh)
    (xs3,) = _tc_call(_mid1_body, [((N_PAD, D_OUT), f32)], 7)(
        a2, xs2, degp, b2, g2, be2, W3)
    p3 = _make_agg_part_kernel()(ileave(xs3), srcw, dstw, zerosh)
    (out,) = _tc_call(_fin_body, [((N, D_OUT), f32)], 4)(p3, xs3, degp, b3)
    return out
